# baseline (speedup 1.0000x reference)
import functools

import jax
import jax.numpy as jnp
import numpy as np
from jax import lax
from jax.experimental import pallas as pl
from jax.experimental.pallas import tpu as pltpu
from jax.experimental.pallas import tpu_sc as plsc

_B = 4096
_NBR = 197
_NEI = 3
_LOG_2PI = float(np.log(2.0 * np.pi))

_L = 16
_NC, _NS = 2, 16
_NW = _NC * _NS
_TPW = _B // _NW
_TV = _TPW // _L

_BR = 32
_NCH = _NBR // _BR
_BRT = _NBR - _NCH * _BR
_SLAB = _NEI * _BR * _TPW

_CHUNKS = [(i * _BR, _BR) for i in range(_NCH)] + [(_NCH * _BR, _BRT)]


def _body(idx_hbm, eps_hbm, wm_hbm, ws_hbm, samp_hbm, logq_hbm,
          idx3_v, idx1_v0, idx1_v1, rows_m0, rows_m1, rows_s0, rows_s1,
          eps_v, samp_v, acc_v,
          sem_stage, sem_rows0, sem_rows1):
  c = lax.axis_index("c")
  s = lax.axis_index("s")
  wid = s * _NC + c
  tbase = wid * _TPW

  for k in range(_TV):
    acc_v[pl.ds(k * _L, _L)] = jnp.zeros((_L,), jnp.float32)

  def stage_start(g):
    br0, br_n = _CHUNKS[g]
    b = g % 2
    pltpu.async_copy(
        idx_hbm.at[:, pl.ds(br0, br_n), pl.ds(tbase, _TPW)],
        idx3_v.at[b, :, pl.ds(0, br_n)], sem_stage)
    pltpu.async_copy(
        eps_hbm.at[pl.ds(br0, br_n), pl.ds(tbase, _TPW)],
        eps_v.at[b, pl.ds(0, br_n)], sem_stage)

  def stage_wait(g):
    br0, br_n = _CHUNKS[g]
    b = g % 2
    pltpu.make_async_copy(
        idx_hbm.at[:, pl.ds(br0, br_n), pl.ds(tbase, _TPW)],
        idx3_v.at[b, :, pl.ds(0, br_n)], sem_stage).wait()
    pltpu.make_async_copy(
        eps_hbm.at[pl.ds(br0, br_n), pl.ds(tbase, _TPW)],
        eps_v.at[b, pl.ds(0, br_n)], sem_stage).wait()

  def flatten_and_fire(g):
    _, br_n = _CHUNKS[g]
    b = g % 2
    slab = _NEI * br_n * _TPW
    idx1_v = idx1_v0 if b == 0 else idx1_v1
    rows_m = rows_m0 if b == 0 else rows_m1
    rows_s = rows_s0 if b == 0 else rows_s1

    def flat_br(n):
      def inner(br, cc):
        r = n * br_n + br
        for k in range(_TV):
          idx1_v[pl.ds(r * _TPW + k * _L, _L)] = (
              idx3_v[b, n, br, pl.ds(k * _L, _L)])
        return cc
      return inner
    for n in range(_NEI):
      lax.fori_loop(0, br_n, flat_br(n), 0)

    sem_rows = sem_rows0 if b == 0 else sem_rows1
    q = slab // 4
    for ss in range(4):
      pltpu.async_copy(wm_hbm.at[idx1_v.at[pl.ds(ss * q, q)]],
                       rows_m.at[pl.ds(ss * q, q)], sem_rows)
      pltpu.async_copy(ws_hbm.at[idx1_v.at[pl.ds(ss * q, q)]],
                       rows_s.at[pl.ds(ss * q, q)], sem_rows)

  def drain_and_compute(g):
    br0, br_n = _CHUNKS[g]
    b = g % 2
    slab = _NEI * br_n * _TPW
    sem_rows = sem_rows0 if b == 0 else sem_rows1
    idx1_v = idx1_v0 if b == 0 else idx1_v1
    rows_m = rows_m0 if b == 0 else rows_m1
    rows_s = rows_s0 if b == 0 else rows_s1
    q = slab // 4
    for ss in range(4):
      pltpu.make_async_copy(wm_hbm.at[idx1_v.at[pl.ds(ss * q, q)]],
                            rows_m.at[pl.ds(ss * q, q)], sem_rows).wait()
      pltpu.make_async_copy(ws_hbm.at[idx1_v.at[pl.ds(ss * q, q)]],
                            rows_s.at[pl.ds(ss * q, q)], sem_rows).wait()

    def comp_row(br, cc):
      o0 = br * _TPW
      o1 = (br_n + br) * _TPW
      o2 = (2 * br_n + br) * _TPW
      for k in range(_TV):
        kk = k * _L
        m = (rows_m[pl.ds(o0 + kk, _L)] + rows_m[pl.ds(o1 + kk, _L)]
             + rows_m[pl.ds(o2 + kk, _L)])
        sd = (rows_s[pl.ds(o0 + kk, _L)] + rows_s[pl.ds(o1 + kk, _L)]
              + rows_s[pl.ds(o2 + kk, _L)])
        e = eps_v[b, br, pl.ds(kk, _L)]
        samp_v[b, br, pl.ds(kk, _L)] = e * jnp.exp(sd) + m
        acc_v[pl.ds(kk, _L)] = acc_v[pl.ds(kk, _L)] - 0.5 * e * e - sd
      return cc
    lax.fori_loop(0, br_n, comp_row, 0)

    pltpu.sync_copy(
        samp_v.at[b, pl.ds(0, br_n)],
        samp_hbm.at[pl.ds(br0, br_n), pl.ds(tbase, _TPW)])

  n_chunks = len(_CHUNKS)
  stage_start(0)
  stage_wait(0)
  flatten_and_fire(0)
  for g in range(n_chunks):
    if g + 1 < n_chunks:
      stage_start(g + 1)
      stage_wait(g + 1)
      flatten_and_fire(g + 1)
    drain_and_compute(g)

  for k in range(_TV):
    acc_v[pl.ds(k * _L, _L)] = (
        acc_v[pl.ds(k * _L, _L)] - 0.5 * _NBR * _LOG_2PI)
  pltpu.sync_copy(acc_v, logq_hbm.at[pl.ds(tbase, _TPW)])


_encoder = functools.partial(
    pl.kernel,
    out_type=[jax.ShapeDtypeStruct((_NBR, _B), jnp.float32),
              jax.ShapeDtypeStruct((_B,), jnp.float32)],
    mesh=plsc.VectorSubcoreMesh(core_axis_name="c", subcore_axis_name="s"),
    compiler_params=pltpu.CompilerParams(
        needs_layout_passes=False, use_tc_tiling_on_sc=True
    ),
    scratch_types=[
        pltpu.VMEM((2, _NEI, _BR, _TPW), jnp.int32),
        pltpu.VMEM((_SLAB,), jnp.int32),
        pltpu.VMEM((_SLAB,), jnp.int32),
        pltpu.VMEM((_SLAB,), jnp.float32),
        pltpu.VMEM((_SLAB,), jnp.float32),
        pltpu.VMEM((_SLAB,), jnp.float32),
        pltpu.VMEM((_SLAB,), jnp.float32),
        pltpu.VMEM((2, _BR, _TPW), jnp.float32),
        pltpu.VMEM((2, _BR, _TPW), jnp.float32),
        pltpu.VMEM((_TPW,), jnp.float32),
        pltpu.SemaphoreType.DMA,
        pltpu.SemaphoreType.DMA,
        pltpu.SemaphoreType.DMA,
    ],
)(_body)


@jax.jit
def kernel(neigh_ss_idxes, eps, W):
  idx_t = jnp.transpose(neigh_ss_idxes, (2, 1, 0))
  eps_t = eps.T
  w_mean = W[:, 0]
  w_std = W[:, 1]
  samp_t, logq = _encoder(idx_t, eps_t, w_mean, w_std)
  return samp_t.T, logq, neigh_ss_idxes
